# baseline (speedup 1.0000x reference)
import functools

import jax
import jax.numpy as jnp
from jax import lax
from jax.experimental import layout as jax_layout
from jax.experimental import pallas as pl
from jax.experimental.pallas import tpu as pltpu
from jax.experimental.pallas import tpu_sc as plsc

D = 64
W = 10
K1 = 21
NC, NS, L = 2, 16, 16
NW = NC * NS
CH = 16

CTX_IW = 80
OUT_IW = 112


def _sc_logits(ctx_idx, out_idx, input_table, output_table, B):
    b_per_w = B // NW
    nchunk = b_per_w // CH
    crpc = CH * W // CTX_IW
    orpc = CH * K1 // OUT_IW
    mesh = plsc.VectorSubcoreMesh(core_axis_name="c", subcore_axis_name="s")

    @functools.partial(
        pl.kernel,
        out_type=jax.ShapeDtypeStruct((K1, B), jnp.float32),
        mesh=mesh,
        scratch_types=[
            pltpu.VMEM((nchunk * crpc, CTX_IW), jnp.int32),
            pltpu.VMEM((nchunk * orpc, OUT_IW), jnp.int32),
            [pltpu.VMEM((CH * W, D), jnp.float32) for _ in range(2)],
            [pltpu.VMEM((CH * K1, D), jnp.float32) for _ in range(2)],
            pltpu.VMEM((K1, b_per_w), jnp.float32),
            [pltpu.SemaphoreType.DMA for _ in range(2)],
        ],
        compiler_params=pltpu.CompilerParams(
            needs_layout_passes=False, use_tc_tiling_on_sc=False),
    )
    def body(ctx_idx_hbm, out_idx_hbm, itab_hbm, otab_hbm, out_hbm,
             cidx_v, oidx_v, crows_v, orows_v, lg_v, sems):
        wid = lax.axis_index("s") * NC + lax.axis_index("c")
        iota = lax.iota(jnp.int32, L)
        row_c = iota * W
        row_o = iota * K1

        pltpu.sync_copy(
            ctx_idx_hbm.at[pl.ds(wid * nchunk * crpc, nchunk * crpc)], cidx_v)
        pltpu.sync_copy(
            out_idx_hbm.at[pl.ds(wid * nchunk * orpc, nchunk * orpc)], oidx_v)

        def issue(c, buf):
            for j in range(crpc):
                pltpu.async_copy(
                    itab_hbm.at[cidx_v.at[c * crpc + j]],
                    crows_v[buf].at[pl.ds(j * CTX_IW, CTX_IW)], sems[buf])
            for j in range(orpc):
                pltpu.async_copy(
                    otab_hbm.at[oidx_v.at[c * orpc + j]],
                    orows_v[buf].at[pl.ds(j * OUT_IW, OUT_IW)], sems[buf])

        def wait(c, buf):
            for j in range(crpc):
                pltpu.make_async_copy(
                    itab_hbm.at[cidx_v.at[c * crpc + j]],
                    crows_v[buf].at[pl.ds(j * CTX_IW, CTX_IW)],
                    sems[buf]).wait()
            for j in range(orpc):
                pltpu.make_async_copy(
                    otab_hbm.at[oidx_v.at[c * orpc + j]],
                    orows_v[buf].at[pl.ds(j * OUT_IW, OUT_IW)],
                    sems[buf]).wait()

        def compute(c, buf):
            zeros = tuple(jnp.zeros((L,), jnp.float32) for _ in range(K1))

            @plsc.parallel_loop(0, D, unroll=8, carry=zeros)
            def accs(d, acc_in):
                dd = (iota + d) & (D - 1)
                s = plsc.load_gather(crows_v[buf], [row_c, dd])
                for w in range(1, W):
                    s = s + plsc.load_gather(crows_v[buf], [row_c + w, dd])
                out = []
                for k in range(K1):
                    r = plsc.load_gather(orows_v[buf], [row_o + k, dd])
                    out.append(acc_in[k] + s * r)
                return tuple(out)

            for k in range(K1):
                lg_v[k, pl.ds(c * CH, CH)] = accs[k] * (1.0 / W)

        issue(0, 0)

        def pair_body(t, carry):
            c0 = 2 * t
            issue(c0 + 1, 1)
            wait(c0, 0)
            compute(c0, 0)

            @pl.when(t < nchunk // 2 - 1)
            def _():
                issue(c0 + 2, 0)

            wait(c0 + 1, 1)
            compute(c0 + 1, 1)
            return carry

        lax.fori_loop(0, nchunk // 2, pair_body, 0)
        pltpu.sync_copy(lg_v, out_hbm.at[:, pl.ds(wid * b_per_w, b_per_w)])

    return body(ctx_idx, out_idx, input_table, output_table)


def _loss_body(x_ref, o_ref):
    x = x_ref[...]
    rows = lax.broadcasted_iota(jnp.int32, x.shape, 0)
    lab = (rows < 128).astype(jnp.float32)
    loss = jnp.maximum(x, 0.0) - x * lab + jnp.log1p(jnp.exp(-jnp.abs(x)))
    o_ref[0, 0] = jnp.sum(loss) / (x.shape[0] * x.shape[1])


def kernel(context, target, negatives, input_table, output_table):
    B = context.shape[0]
    ctx_idx = context.astype(jnp.int32).reshape(B * W // CTX_IW, CTX_IW)
    out_idx = jnp.concatenate(
        [target.astype(jnp.int32), negatives.astype(jnp.int32)], axis=1
    ).reshape(B * K1 // OUT_IW, OUT_IW)
    lin = jax_layout.Layout(major_to_minor=(0, 1), tiling=())
    input_table, output_table = jax_layout.with_layout_constraint(
        (input_table, output_table), (lin, lin))
    logits_t = _sc_logits(ctx_idx, out_idx, input_table, output_table, B)
    x = logits_t.reshape(K1 * (B // 128), 128)
    loss = pl.pallas_call(
        _loss_body,
        out_shape=jax.ShapeDtypeStruct((1, 1), jnp.float32),
        out_specs=pl.BlockSpec(memory_space=pltpu.SMEM),
    )(x)
    return loss[0, 0]


# speedup vs baseline: 1.0748x; 1.0748x over previous
import functools

import jax
import jax.numpy as jnp
from jax import lax
from jax.experimental import layout as jax_layout
from jax.experimental import pallas as pl
from jax.experimental.pallas import tpu as pltpu
from jax.experimental.pallas import tpu_sc as plsc

D = 64
W = 10
K1 = 21
NC, NS, L = 2, 16, 16
NW = NC * NS
CH = 16

CTX_IW = 80
OUT_IW = 112


def _sc_logits(ctx_idx, out_idx, input_table, output_table, B):
    b_per_w = B // NW
    nchunk = b_per_w // CH
    crpc = CH * W // CTX_IW
    orpc = CH * K1 // OUT_IW
    mesh = plsc.VectorSubcoreMesh(core_axis_name="c", subcore_axis_name="s")

    @functools.partial(
        pl.kernel,
        out_type=jax.ShapeDtypeStruct((K1, B), jnp.float32),
        mesh=mesh,
        scratch_types=[
            pltpu.VMEM((nchunk * crpc, CTX_IW), jnp.int32),
            pltpu.VMEM((nchunk * orpc, OUT_IW), jnp.int32),
            [pltpu.VMEM((CH * W, D), jnp.float32) for _ in range(2)],
            [pltpu.VMEM((CH * K1, D), jnp.float32) for _ in range(2)],
            pltpu.VMEM((K1, b_per_w), jnp.float32),
            [pltpu.SemaphoreType.DMA for _ in range(2)],
        ],
        compiler_params=pltpu.CompilerParams(
            needs_layout_passes=False, use_tc_tiling_on_sc=False),
    )
    def body(ctx_idx_hbm, out_idx_hbm, itab_hbm, otab_hbm, out_hbm,
             cidx_v, oidx_v, crows_v, orows_v, lg_v, sems):
        wid = lax.axis_index("s") * NC + lax.axis_index("c")
        iota = lax.iota(jnp.int32, L)
        row_c = iota * W
        row_o = iota * K1

        pltpu.sync_copy(
            ctx_idx_hbm.at[pl.ds(wid * nchunk * crpc, nchunk * crpc)], cidx_v)
        pltpu.sync_copy(
            out_idx_hbm.at[pl.ds(wid * nchunk * orpc, nchunk * orpc)], oidx_v)

        def issue(c, buf):
            for j in range(crpc):
                pltpu.async_copy(
                    itab_hbm.at[cidx_v.at[c * crpc + j]],
                    crows_v[buf].at[pl.ds(j * CTX_IW, CTX_IW)], sems[buf])
            for j in range(orpc):
                pltpu.async_copy(
                    otab_hbm.at[oidx_v.at[c * orpc + j]],
                    orows_v[buf].at[pl.ds(j * OUT_IW, OUT_IW)], sems[buf])

        def wait(c, buf):
            for j in range(crpc):
                pltpu.make_async_copy(
                    itab_hbm.at[cidx_v.at[c * crpc + j]],
                    crows_v[buf].at[pl.ds(j * CTX_IW, CTX_IW)],
                    sems[buf]).wait()
            for j in range(orpc):
                pltpu.make_async_copy(
                    otab_hbm.at[oidx_v.at[c * orpc + j]],
                    orows_v[buf].at[pl.ds(j * OUT_IW, OUT_IW)],
                    sems[buf]).wait()

        def compute(c, buf):
            zeros = tuple(jnp.zeros((L,), jnp.float32) for _ in range(K1))

            @plsc.parallel_loop(0, D, unroll=4, carry=zeros)
            def accs(d, acc_in):
                dd = (iota + d) & (D - 1)
                s = plsc.load_gather(crows_v[buf], [row_c, dd])
                for w in range(1, W):
                    s = s + plsc.load_gather(crows_v[buf], [row_c + w, dd])
                out = []
                for k in range(K1):
                    r = plsc.load_gather(orows_v[buf], [row_o + k, dd])
                    out.append(acc_in[k] + s * r)
                return tuple(out)

            for k in range(K1):
                lg_v[k, pl.ds(c * CH, CH)] = accs[k] * (1.0 / W)

        issue(0, 0)

        def pair_body(t, carry):
            c0 = 2 * t
            issue(c0 + 1, 1)
            wait(c0, 0)
            compute(c0, 0)

            @pl.when(t < nchunk // 2 - 1)
            def _():
                issue(c0 + 2, 0)

            wait(c0 + 1, 1)
            compute(c0 + 1, 1)
            return carry

        lax.fori_loop(0, nchunk // 2, pair_body, 0)
        pltpu.sync_copy(lg_v, out_hbm.at[:, pl.ds(wid * b_per_w, b_per_w)])

    return body(ctx_idx, out_idx, input_table, output_table)


def _loss_body(x_ref, o_ref):
    x = x_ref[...]
    rows = lax.broadcasted_iota(jnp.int32, x.shape, 0)
    lab = (rows < 128).astype(jnp.float32)
    loss = jnp.maximum(x, 0.0) - x * lab + jnp.log1p(jnp.exp(-jnp.abs(x)))
    o_ref[0, 0] = jnp.sum(loss) / (x.shape[0] * x.shape[1])


def kernel(context, target, negatives, input_table, output_table):
    B = context.shape[0]
    ctx_idx = context.astype(jnp.int32).reshape(B * W // CTX_IW, CTX_IW)
    out_idx = jnp.concatenate(
        [target.astype(jnp.int32), negatives.astype(jnp.int32)], axis=1
    ).reshape(B * K1 // OUT_IW, OUT_IW)
    lin = jax_layout.Layout(major_to_minor=(0, 1), tiling=())
    input_table, output_table = jax_layout.with_layout_constraint(
        (input_table, output_table), (lin, lin))
    logits_t = _sc_logits(ctx_idx, out_idx, input_table, output_table, B)
    x = logits_t.reshape(K1 * (B // 128), 128)
    loss = pl.pallas_call(
        _loss_body,
        out_shape=jax.ShapeDtypeStruct((1, 1), jnp.float32),
        out_specs=pl.BlockSpec(memory_space=pltpu.SMEM),
    )(x)
    return loss[0, 0]
